# baseline (speedup 1.0000x reference)
import jax
import jax.numpy as jnp
from jax import lax
from jax.experimental import pallas as pl
from jax.experimental.pallas import tpu as pltpu
from jax.experimental.pallas import tpu_sc as plsc

_H = 320
_W = 320
_CONTRAST = 4
_COIL = 12
_SLOPE = 5.0
_CENTER = 10
_R = 4.0
_NC = 2
_NS = 16
_HALF = _COIL // 2
_RCH = 80
_NCHUNK = _H // _RCH
_STEPS = _HALF * _NCHUNK


def _mask_body(w_ref, noise_ref, m_ref, onem_ref):
    w = w_ref[0]
    prob = jax.nn.sigmoid(w * _SLOPE)
    rows = jax.lax.broadcasted_iota(jnp.int32, (_H, _W), 0)
    cols = jax.lax.broadcasted_iota(jnp.int32, (_H, _W), 1)
    cy0, cy1 = _H // 2 - _CENTER // 2, _H // 2 + _CENTER // 2
    cx0, cx1 = _W // 2 - _CENTER // 2, _W // 2 + _CENTER // 2
    in_center = (rows >= cy0) & (rows < cy1) & (cols >= cx0) & (cols < cx1)
    p = jnp.where(in_center, 0.0, prob)
    s = jnp.sum(p)
    total = _H * _W / _R - _CENTER ** 2
    p_over = p * (total / s)
    inv_total = _H * _W * (1.0 - 1.0 / _R)
    inv_sum = _H * _W - s - _CENTER ** 2
    p_under = 1.0 - (1.0 - p) * (inv_total / inv_sum)
    p_new = jnp.where(s > total, p_over, p_under)
    p_new = jnp.where(in_center, 1.0, p_new)
    m = (p_new - noise_ref[0] >= 0.0).astype(jnp.float32)
    m_ref[0] = m
    onem_ref[0] = 1.0 - m


def _lambda_body(m_ref, im_ref, lam_ref):
    lam_ref[0] = im_ref[0] * m_ref[0][None, :, :]


def _sc_inverse_body(im_hbm, onem_hbm, inv_hbm, mask_v, a_v, b_v,
                     isem_a, isem_b, osem_a, osem_b):
    c = lax.axis_index("c")
    s = lax.axis_index("s")
    wid = s * _NC + c
    bc = wid // 2
    coil0 = (wid % 2) * _HALF
    plane0 = bc * _COIL + coil0

    bufs = (a_v, b_v)
    isems = (isem_a, isem_b)
    osems = (osem_a, osem_b)

    def src_slice(step):
        chunk, k = divmod(step, _HALF)
        return plane0 + k, chunk * _RCH

    def start_in(step):
        i = step % 2
        p, r0 = src_slice(step)
        return pltpu.make_async_copy(
            im_hbm.at[p, pl.ds(r0, _RCH), :], bufs[i], isems[i])

    def start_out(step):
        i = step % 2
        p, r0 = src_slice(step)
        return pltpu.make_async_copy(
            bufs[i], inv_hbm.at[p, pl.ds(r0, _RCH), :], osems[i])

    h = start_in(0)
    h.start()
    in_h = [h, None]
    out_h = [None, None]
    for step in range(_STEPS):
        i = step % 2
        if step + 1 < _STEPS:
            j = (step + 1) % 2
            if out_h[j] is not None:
                out_h[j].wait()
                out_h[j] = None
            nh = start_in(step + 1)
            nh.start()
            in_h[j] = nh
        if step % _HALF == 0:
            chunk = step // _HALF
            pltpu.sync_copy(
                onem_hbm.at[bc, pl.ds(chunk * _RCH, _RCH), :], mask_v)
        in_h[i].wait()
        buf = bufs[i]

        def row_body(r, _):
            for c16 in range(_W // 16):
                sl = pl.ds(c16 * 16, 16)
                buf[r, sl] = buf[r, sl] * mask_v[r, sl]
            return 0

        lax.fori_loop(0, _RCH, row_body, 0)
        oh = start_out(step)
        oh.start()
        out_h[i] = oh
    for oh in out_h:
        if oh is not None:
            oh.wait()


def kernel(undersampled_k, initial_mask, sampling_weights):
    batch = undersampled_k.shape[0]
    with jax.ensure_compile_time_eval():
        noise = jax.random.uniform(
            jax.random.key(42), (batch, _CONTRAST, _H, _W), dtype=jnp.float32
        )
    bc = batch * _CONTRAST
    im4 = initial_mask.reshape(bc, _COIL, _H, _W)
    noise3 = noise.reshape(bc, _H, _W)

    m16, onem16 = pl.pallas_call(
        _mask_body,
        grid=(bc,),
        in_specs=[
            pl.BlockSpec((1, _H, _W), lambda i: (i % _CONTRAST, 0, 0)),
            pl.BlockSpec((1, _H, _W), lambda i: (i, 0, 0)),
        ],
        out_specs=[
            pl.BlockSpec((1, _H, _W), lambda i: (i, 0, 0)),
            pl.BlockSpec((1, _H, _W), lambda i: (i, 0, 0)),
        ],
        out_shape=[
            jax.ShapeDtypeStruct((bc, _H, _W), jnp.float32),
            jax.ShapeDtypeStruct((bc, _H, _W), jnp.float32),
        ],
    )(sampling_weights, noise3)

    lam = pl.pallas_call(
        _lambda_body,
        grid=(bc,),
        in_specs=[
            pl.BlockSpec((1, _H, _W), lambda i: (i, 0, 0)),
            pl.BlockSpec((1, _COIL, _H, _W), lambda i: (i, 0, 0, 0)),
        ],
        out_specs=pl.BlockSpec((1, _COIL, _H, _W), lambda i: (i, 0, 0, 0)),
        out_shape=jax.ShapeDtypeStruct((bc, _COIL, _H, _W), jnp.float32),
    )(m16, im4)

    im3 = initial_mask.reshape(bc * _COIL, _H, _W)
    mesh = plsc.VectorSubcoreMesh(
        core_axis_name="c", subcore_axis_name="s",
        num_cores=_NC, num_subcores=_NS,
    )
    inv = pl.kernel(
        _sc_inverse_body,
        out_type=jax.ShapeDtypeStruct((bc * _COIL, _H, _W), jnp.float32),
        mesh=mesh,
        scratch_types=[
            pltpu.VMEM((_RCH, _W), jnp.float32),
            pltpu.VMEM((_RCH, _W), jnp.float32),
            pltpu.VMEM((_RCH, _W), jnp.float32),
            pltpu.SemaphoreType.DMA,
            pltpu.SemaphoreType.DMA,
            pltpu.SemaphoreType.DMA,
            pltpu.SemaphoreType.DMA,
        ],
    )(im3, onem16)

    shape5 = (batch, _CONTRAST, _COIL, _H, _W)
    return (lam.reshape(shape5), inv.reshape(shape5))


# speedup vs baseline: 1.8356x; 1.8356x over previous
import jax
import jax.numpy as jnp
from jax.experimental import pallas as pl
from jax.experimental.pallas import tpu as pltpu

_H = 320
_W = 320
_CONTRAST = 4
_COIL = 12
_SLOPE = 5.0
_CENTER = 10
_R = 4.0


def _fused_body(w_ref, noise_ref, im_ref, lam_ref, inv_ref):
    w = w_ref[0]
    prob = jax.nn.sigmoid(w * _SLOPE)
    rows = jax.lax.broadcasted_iota(jnp.int32, (_H, _W), 0)
    cols = jax.lax.broadcasted_iota(jnp.int32, (_H, _W), 1)
    cy0, cy1 = _H // 2 - _CENTER // 2, _H // 2 + _CENTER // 2
    cx0, cx1 = _W // 2 - _CENTER // 2, _W // 2 + _CENTER // 2
    in_center = (rows >= cy0) & (rows < cy1) & (cols >= cx0) & (cols < cx1)
    p = jnp.where(in_center, 0.0, prob)
    s = jnp.sum(p)
    total = _H * _W / _R - _CENTER ** 2
    p_over = p * (total / s)
    inv_total = _H * _W * (1.0 - 1.0 / _R)
    inv_sum = _H * _W - s - _CENTER ** 2
    p_under = 1.0 - (1.0 - p) * (inv_total / inv_sum)
    p_new = jnp.where(s > total, p_over, p_under)
    p_new = jnp.where(in_center, 1.0, p_new)
    m = (p_new - noise_ref[0] >= 0.0).astype(jnp.float32)
    im = im_ref[0]
    lam = im * m[None, :, :]
    lam_ref[0] = lam
    inv_ref[0] = im - lam


def kernel(undersampled_k, initial_mask, sampling_weights):
    batch = undersampled_k.shape[0]
    with jax.ensure_compile_time_eval():
        noise = jax.random.uniform(
            jax.random.key(42), (batch, _CONTRAST, _H, _W), dtype=jnp.float32
        )
    bc = batch * _CONTRAST
    im = initial_mask.reshape(bc, _COIL, _H, _W)
    noise_f = noise.reshape(bc, _H, _W)

    lam, inv = pl.pallas_call(
        _fused_body,
        grid=(bc,),
        in_specs=[
            pl.BlockSpec((1, _H, _W), lambda i: (i % _CONTRAST, 0, 0)),
            pl.BlockSpec((1, _H, _W), lambda i: (i, 0, 0)),
            pl.BlockSpec((1, _COIL, _H, _W), lambda i: (i, 0, 0, 0)),
        ],
        out_specs=[
            pl.BlockSpec((1, _COIL, _H, _W), lambda i: (i, 0, 0, 0)),
            pl.BlockSpec((1, _COIL, _H, _W), lambda i: (i, 0, 0, 0)),
        ],
        out_shape=[
            jax.ShapeDtypeStruct((bc, _COIL, _H, _W), jnp.float32),
            jax.ShapeDtypeStruct((bc, _COIL, _H, _W), jnp.float32),
        ],
        compiler_params=pltpu.CompilerParams(
            dimension_semantics=("parallel",),
        ),
    )(sampling_weights, noise_f, im)

    shape5 = (batch, _CONTRAST, _COIL, _H, _W)
    return (lam.reshape(shape5), inv.reshape(shape5))
